# Initial kernel scaffold; baseline (speedup 1.0000x reference)
import jax
import jax.numpy as jnp
from jax.experimental import pallas as pl


def kernel(nodes_feat, edges_feat, nodes_num_norm_sqrt, edges_num_norm_sqrt, edge_index, params):
    raise NotImplementedError("write your pallas kernel here")


# speedup vs baseline: 1.1046x; 1.1046x over previous
import functools

import jax
import jax.numpy as jnp
from jax import lax
from jax.experimental import pallas as pl
from jax.experimental.pallas import tpu as pltpu
from jax.experimental.pallas import tpu_sc as plsc

N = 10000
E = 320000
D = 128
H = 64
NC = 2
NS = 16
TILE_E = E // NS
CH = 80
NCH = TILE_E // CH
N_PAD = 10240
ZR = 128
TB_N = 400
TB_E = 512
F32 = jnp.float32


def _sc_edge_body(emit_enew, *refs):
    if emit_enew:
        (db_hbm, eh_hbm, base_hbm, src_hbm, dst_hbm,
         numden_hbm, enew_hbm,
         acc_sh, srcb, dstb, idb, ieb, dbbuf, ehbuf, basebuf, valbuf,
         enewbuf, sem1, sem2) = refs
    else:
        (db_hbm, eh_hbm, base_hbm, src_hbm, dst_hbm,
         numden_hbm,
         acc_sh, srcb, dstb, idb, ieb, dbbuf, ehbuf, basebuf, valbuf,
         enewbuf, sem1, sem2) = refs
        enew_hbm = None

    c = lax.axis_index("c")
    s = lax.axis_index("s")
    row0 = s * (N_PAD // NS)

    def zfill(t, _):
        for j in range(8):
            valbuf[t, pl.ds(j * 16, 16)] = jnp.zeros((16,), F32)
        return 0
    lax.fori_loop(0, CH, zfill, 0)
    for k in range(8):
        pltpu.sync_copy(valbuf, acc_sh.at[pl.ds(row0 + k * CH, CH), :])
    plsc.subcore_barrier()

    ebase = s * TILE_E
    cN = c * N

    def chunk(k, carry):
        e0 = ebase + k * CH
        pltpu.sync_copy(src_hbm.at[pl.ds(e0, CH)], srcb)
        pltpu.sync_copy(dst_hbm.at[pl.ds(e0, CH)], dstb)
        pltpu.sync_copy(base_hbm.at[c, pl.ds(e0, CH), :], basebuf)

        def mkidx(i, _):
            sl = pl.ds(i * 16, 16)
            idb[sl] = srcb[sl] + cN
            ieb[sl] = dstb[sl] + cN
            return 0
        lax.fori_loop(0, CH // 16, mkidx, 0)

        cp1 = pltpu.async_copy(db_hbm.at[idb], dbbuf, sem1)
        cp2 = pltpu.async_copy(eh_hbm.at[ieb], ehbuf, sem2)
        cp1.wait()
        cp2.wait()

        def edge(t, _):
            for j in range(4):
                fl = pl.ds(j * 16, 16)
                fh = pl.ds(64 + j * 16, 16)
                z = basebuf[t, fl] + dbbuf[t, fl] + ehbuf[t, fl]
                if emit_enew:
                    enewbuf[t, fl] = z
                sg = 1.0 / (1.0 + jnp.exp(-z))
                valbuf[t, fl] = sg * dbbuf[t, fh]
                valbuf[t, fh] = sg
            return 0
        lax.fori_loop(0, CH, edge, 0)

        if emit_enew:
            pltpu.sync_copy(enewbuf, enew_hbm.at[c, pl.ds(e0, CH), :])
        pltpu.sync_copy(valbuf, acc_sh.at[dstb], add=True)
        return 0

    lax.fori_loop(0, NCH, chunk, 0)

    plsc.subcore_barrier()
    for k in range(5):
        r0 = row0 + k * ZR
        pltpu.sync_copy(acc_sh.at[pl.ds(r0, ZR), :],
                        numden_hbm.at[c, pl.ds(r0, ZR), :])


def _make_sc_edge(emit_enew):
    if emit_enew:
        out_type = (jax.ShapeDtypeStruct((NC, N_PAD, D), F32),
                    jax.ShapeDtypeStruct((NC, E, H), F32))
    else:
        out_type = jax.ShapeDtypeStruct((NC, N_PAD, D), F32)
    mesh = plsc.VectorSubcoreMesh(core_axis_name="c", subcore_axis_name="s",
                                  num_cores=NC, num_subcores=NS)
    return pl.kernel(
        functools.partial(_sc_edge_body, emit_enew),
        out_type=out_type,
        mesh=mesh,
        compiler_params=pltpu.CompilerParams(use_tc_tiling_on_sc=False),
        scratch_types=[
            pltpu.VMEM_SHARED((N_PAD, D), F32),
            pltpu.VMEM((CH,), jnp.int32),
            pltpu.VMEM((CH,), jnp.int32),
            pltpu.VMEM((CH,), jnp.int32),
            pltpu.VMEM((CH,), jnp.int32),
            pltpu.VMEM((CH, D), F32),
            pltpu.VMEM((CH, H), F32),
            pltpu.VMEM((CH, H), F32),
            pltpu.VMEM((CH, D), F32),
            pltpu.VMEM((CH, H), F32),
            pltpu.SemaphoreType.DMA,
            pltpu.SemaphoreType.DMA,
        ],
        name="sc_edge_pass" + ("_l1" if emit_enew else "_l2"),
    )


_sc_edge_l1 = _make_sc_edge(True)
_sc_edge_l2 = _make_sc_edge(False)


def _mm_body(x_ref, w_ref, b_ref, o_ref):
    o_ref[...] = jnp.dot(x_ref[...], w_ref[...],
                         preferred_element_type=F32) + b_ref[...]


def _mm(x, w, b):
    n, din = x.shape
    dout = w.shape[1]
    return pl.pallas_call(
        _mm_body,
        grid=(n // TB_N,),
        in_specs=[pl.BlockSpec((TB_N, din), lambda i: (i, 0)),
                  pl.BlockSpec((din, dout), lambda i: (0, 0)),
                  pl.BlockSpec((1, dout), lambda i: (0, 0))],
        out_specs=pl.BlockSpec((TB_N, dout), lambda i: (i, 0)),
        out_shape=jax.ShapeDtypeStruct((n, dout), F32),
    )(x, w, b.reshape(1, dout))


def _outer_body(ef_ref, uv_ref, o_ref):
    t = ef_ref[...] * uv_ref[0:1, :] + uv_ref[1:2, :]
    o_ref[0] = t[:, :H]
    o_ref[1] = t[:, H:]


def _outer(ef, u, v):
    uv = jnp.stack([u, v])
    return pl.pallas_call(
        _outer_body,
        grid=(E // TB_E,),
        in_specs=[pl.BlockSpec((TB_E, 1), lambda i: (i, 0)),
                  pl.BlockSpec((2, D), lambda i: (0, 0))],
        out_specs=pl.BlockSpec((NC, TB_E, H), lambda i: (0, i, 0)),
        out_shape=jax.ShapeDtypeStruct((NC, E, H), F32),
    )(ef, uv)


def _estats_body(en_ref, sn_ref, o_ref):
    @pl.when(pl.program_id(0) == 0)
    def _init():
        o_ref[...] = jnp.zeros_like(o_ref)
    eb = jnp.concatenate([en_ref[0], en_ref[1]], axis=1) * sn_ref[...]
    o_ref[...] += jnp.stack([jnp.sum(eb, axis=0), jnp.sum(eb * eb, axis=0)])


def _estats(enew, sne):
    return pl.pallas_call(
        _estats_body,
        grid=(E // TB_E,),
        in_specs=[pl.BlockSpec((NC, TB_E, H), lambda i: (0, i, 0)),
                  pl.BlockSpec((TB_E, 1), lambda i: (i, 0))],
        out_specs=pl.BlockSpec((2, D), lambda i: (0, 0)),
        out_shape=jax.ShapeDtypeStruct((2, D), F32),
    )(enew, sne)


def _ztrans_body(en_ref, sn_ref, ef_ref, c2_ref, aux_ref, o_ref):
    eb = jnp.concatenate([en_ref[0], en_ref[1]], axis=1) * sn_ref[...]
    r = jnp.maximum(eb * aux_ref[0:1, :] + aux_ref[1:2, :], 0.0)
    zc = (jnp.dot(r, c2_ref[...], preferred_element_type=F32)
          + ef_ref[...] * aux_ref[2:3, :] + aux_ref[3:4, :])
    o_ref[0] = zc[:, :H]
    o_ref[1] = zc[:, H:]


def _ztransform(enew, sne, ef, c2w, ge, bbe, u2, v2c):
    aux = jnp.stack([ge, bbe, u2, v2c])
    return pl.pallas_call(
        _ztrans_body,
        grid=(E // TB_E,),
        in_specs=[pl.BlockSpec((NC, TB_E, H), lambda i: (0, i, 0)),
                  pl.BlockSpec((TB_E, 1), lambda i: (i, 0)),
                  pl.BlockSpec((TB_E, 1), lambda i: (i, 0)),
                  pl.BlockSpec((D, D), lambda i: (0, 0)),
                  pl.BlockSpec((4, D), lambda i: (0, 0))],
        out_specs=pl.BlockSpec((NC, TB_E, H), lambda i: (0, i, 0)),
        out_shape=jax.ShapeDtypeStruct((NC, E, H), F32),
    )(enew, sne, ef, c2w, aux)


def _hstats_body(nd_ref, ah_ref, snn_ref, hb_ref, st_ref):
    @pl.when(pl.program_id(0) == 0)
    def _init():
        st_ref[...] = jnp.zeros_like(st_ref)
    num = jnp.concatenate([nd_ref[0, :, :H], nd_ref[1, :, :H]], axis=1)
    den = jnp.concatenate([nd_ref[0, :, H:], nd_ref[1, :, H:]], axis=1)
    m = (ah_ref[...] + num / (den + 1e-6)) * snn_ref[...]
    hb_ref[...] = m
    st_ref[...] += jnp.stack([jnp.sum(m, axis=0), jnp.sum(m * m, axis=0)])


def _hstats(numden, ah, snn):
    return pl.pallas_call(
        _hstats_body,
        grid=(N // TB_N,),
        in_specs=[pl.BlockSpec((NC, TB_N, D), lambda i: (0, i, 0)),
                  pl.BlockSpec((TB_N, D), lambda i: (i, 0)),
                  pl.BlockSpec((TB_N, 1), lambda i: (i, 0))],
        out_specs=[pl.BlockSpec((TB_N, D), lambda i: (i, 0)),
                   pl.BlockSpec((2, D), lambda i: (0, 0))],
        out_shape=[jax.ShapeDtypeStruct((N, D), F32),
                   jax.ShapeDtypeStruct((2, D), F32)],
    )(numden, ah, snn)


def _happly_body(hp_ref, hb_ref, gb_ref, o_ref, cs_ref):
    @pl.when(pl.program_id(0) == 0)
    def _init():
        cs_ref[...] = jnp.zeros_like(cs_ref)
    hn = hp_ref[...] + jnp.maximum(
        hb_ref[...] * gb_ref[0:1, :] + gb_ref[1:2, :], 0.0)
    o_ref[...] = hn
    cs_ref[...] += jnp.sum(hn, axis=0, keepdims=True)


def _happly(hprev, hb, g, b):
    gb = jnp.stack([g, b])
    return pl.pallas_call(
        _happly_body,
        grid=(N // TB_N,),
        in_specs=[pl.BlockSpec((TB_N, D), lambda i: (i, 0)),
                  pl.BlockSpec((TB_N, D), lambda i: (i, 0)),
                  pl.BlockSpec((2, D), lambda i: (0, 0))],
        out_specs=[pl.BlockSpec((TB_N, D), lambda i: (i, 0)),
                   pl.BlockSpec((1, D), lambda i: (0, 0))],
        out_shape=[jax.ShapeDtypeStruct((N, D), F32),
                   jax.ShapeDtypeStruct((1, D), F32)],
    )(hprev, hb, gb)


def _bn_consts(stats, count, gamma, beta, eps=1e-5):
    mu = stats[0] / count
    var = stats[1] / count - mu * mu
    g = gamma / jnp.sqrt(var + eps)
    return g, beta - mu * g


def _split_tables(t4):
    ah = t4[:, :D]
    bh = t4[:, D:2 * D]
    dh = t4[:, 2 * D:3 * D]
    eh = t4[:, 3 * D:]
    db = jnp.concatenate([
        jnp.concatenate([dh[:, :H], bh[:, :H]], axis=1),
        jnp.concatenate([dh[:, H:], bh[:, H:]], axis=1)], axis=0)
    ehh = jnp.concatenate([eh[:, :H], eh[:, H:]], axis=0)
    return ah, db, ehh


def kernel(nodes_feat, edges_feat, nodes_num_norm_sqrt, edges_num_norm_sqrt,
           edge_index, params):
    ef = edges_feat
    sne = edges_num_norm_sqrt
    snn = nodes_num_norm_sqrt
    src = edge_index[0]
    dst = edge_index[1]

    p1, p2 = params['layers']
    w_e = params['emb_e'][0][0]
    b_e = params['emb_e'][1]

    h0 = _mm(nodes_feat, params['emb_h'][0], params['emb_h'][1])

    def layer_tables(h, p):
        wcat = jnp.concatenate([p['A'][0], p['B'][0], p['D'][0], p['E'][0]],
                               axis=1)
        bcat = jnp.concatenate([p['A'][1], p['B'][1], p['D'][1], p['E'][1]])
        return _split_tables(_mm(h, wcat, bcat))

    ah1, db1, ehh1 = layer_tables(h0, p1)
    u1 = w_e @ p1['C'][0]
    v1 = b_e @ p1['C'][0] + p1['C'][1]
    base1 = _outer(ef, u1, v1)
    numden1, enew1 = _sc_edge_l1(db1, ehh1, base1, src, dst)
    hb1, hst1 = _hstats(numden1, ah1, snn)
    g1, b1 = _bn_consts(hst1, N, p1['bn_h'][0], p1['bn_h'][1])
    h1, _ = _happly(h0, hb1, g1, b1)

    est = _estats(enew1, sne)
    ge, bbe = _bn_consts(est, E, p1['bn_e'][0], p1['bn_e'][1])

    ah2, db2, ehh2 = layer_tables(h1, p2)
    u2 = w_e @ p2['C'][0]
    v2c = b_e @ p2['C'][0] + p2['C'][1]
    zc = _ztransform(enew1, sne, ef, p2['C'][0], ge, bbe, u2, v2c)
    numden2 = _sc_edge_l2(db2, ehh2, zc, src, dst)
    hb2, hst2 = _hstats(numden2, ah2, snn)
    g2, b2 = _bn_consts(hst2, N, p2['bn_h'][0], p2['bn_h'][1])
    _, colsum = _happly(h1, hb2, g2, b2)

    return colsum * (1.0 / N)


# speedup vs baseline: 1.2529x; 1.1342x over previous
import functools

import jax
import jax.numpy as jnp
from jax import lax
from jax.experimental import pallas as pl
from jax.experimental.pallas import tpu as pltpu
from jax.experimental.pallas import tpu_sc as plsc

N = 10000
E = 320000
D = 128
H = 64
NC = 2
NS = 16
TILE_E = E // NS
CH = 80
NCH = TILE_E // CH
NPAIR = NCH // 2
NROW = N // NS
TB_N = 400
TB_E = 512
F32 = jnp.float32


def _sc_edge_body(emit_enew, *refs):
    if emit_enew:
        (db_hbm, eh_hbm, base_hbm, sd_hbm,
         num_hbm, den_hbm, enew_hbm,
         accn, accd,
         sdb0, sdb1, idb0, idb1, ieb0, ieb1,
         dbb0, dbb1, ehb0, ehb1, bab0, bab1, enb,
         si0, si1, sgd0, sgd1, sge0, sge1) = refs
    else:
        (db_hbm, eh_hbm, base_hbm, sd_hbm,
         num_hbm, den_hbm,
         accn, accd,
         sdb0, sdb1, idb0, idb1, ieb0, ieb1,
         dbb0, dbb1, ehb0, ehb1, bab0, bab1, enb,
         si0, si1, sgd0, sgd1, sge0, sge1) = refs
        enew_hbm = None

    c = lax.axis_index("c")
    s = lax.axis_index("s")
    row0 = s * NROW
    ebase = s * TILE_E
    cN = c * N

    sdb = (sdb0, sdb1)
    idb = (idb0, idb1)
    ieb = (ieb0, ieb1)
    dbb = (dbb0, dbb1)
    ehb = (ehb0, ehb1)
    bab = (bab0, bab1)
    si = (si0, si1)
    sgd = (sgd0, sgd1)
    sge = (sge0, sge1)

    def zfill(t, _):
        for j in range(4):
            ehb0[t, pl.ds(j * 16, 16)] = jnp.zeros((16,), F32)
        return 0
    lax.fori_loop(0, CH, zfill, 0)
    for acc in (accn, accd):
        for kk in range(7):
            pltpu.sync_copy(ehb0, acc.at[pl.ds(row0 + kk * CH, CH), :])
        pltpu.sync_copy(ehb0.at[pl.ds(0, NROW - 7 * CH)],
                        acc.at[pl.ds(row0 + 7 * CH, NROW - 7 * CH), :])
    plsc.subcore_barrier()

    def issue_in(k, b):
        e0 = ebase + k * CH
        pltpu.async_copy(sd_hbm.at[pl.ds(e0, CH)], sdb[b], si[b])
        pltpu.async_copy(base_hbm.at[c, pl.ds(e0, CH), :], bab[b], si[b])

    def wait_in(k, b):
        e0 = ebase + k * CH
        pltpu.make_async_copy(sd_hbm.at[pl.ds(e0, CH)], sdb[b], si[b]).wait()
        pltpu.make_async_copy(base_hbm.at[c, pl.ds(e0, CH), :], bab[b],
                              si[b]).wait()

    def mkidx(b):
        for j in range(CH // 16):
            sl = pl.ds(j * 16, 16)
            sd = sdb[b][sl]
            idb[b][sl] = (sd & 0xFFFF) + cN
            ieb[b][sl] = lax.shift_right_logical(sd, 16) + cN

    def issue_g(b):
        pltpu.async_copy(db_hbm.at[idb[b]], dbb[b], sgd[b])
        pltpu.async_copy(eh_hbm.at[ieb[b]], ehb[b], sge[b])

    def wait_g(b):
        pltpu.make_async_copy(db_hbm.at[idb[b]], dbb[b], sgd[b]).wait()
        pltpu.make_async_copy(eh_hbm.at[ieb[b]], ehb[b], sge[b]).wait()

    def compute(b):
        dbbb, ehbb, babb = dbb[b], ehb[b], bab[b]

        def edge(t, _):
            for j in range(4):
                fl = pl.ds(j * 16, 16)
                z = babb[t, fl] + dbbb[t, fl] + ehbb[t, fl]
                if emit_enew:
                    enb[t, fl] = z
                sg = 1.0 / (1.0 + jnp.exp(-z))
                ehbb[t, fl] = sg * dbbb[t, pl.ds(64 + j * 16, 16)]
                babb[t, fl] = sg
            return 0
        lax.fori_loop(0, CH, edge, 0, unroll=4)

    def finish(k, b):
        for j in range(CH // 16):
            sl = pl.ds(j * 16, 16)
            ieb[b][sl] = ieb[b][sl] - cN
        if emit_enew:
            e0 = ebase + k * CH
            pltpu.sync_copy(enb, enew_hbm.at[c, pl.ds(e0, CH), :])
        pltpu.sync_copy(ehb[b], accn.at[ieb[b]], add=True)
        pltpu.sync_copy(bab[b], accd.at[ieb[b]], add=True)

    issue_in(0, 0)
    issue_in(1, 1)
    wait_in(0, 0)
    mkidx(0)
    issue_g(0)

    def pair(p, _):
        k0 = 2 * p
        k1 = k0 + 1
        wait_g(0)
        wait_in(k1, 1)
        mkidx(1)
        issue_g(1)
        compute(0)
        finish(k0, 0)

        @pl.when(p < NPAIR - 1)
        def _in0():
            issue_in(k0 + 2, 0)

        wait_g(1)

        @pl.when(p < NPAIR - 1)
        def _g0():
            wait_in(k0 + 2, 0)
            mkidx(0)
            issue_g(0)
        compute(1)
        finish(k1, 1)

        @pl.when(p < NPAIR - 1)
        def _in1():
            issue_in(k1 + 2, 1)
        return 0

    lax.fori_loop(0, NPAIR, pair, 0)

    plsc.subcore_barrier()
    for acc, out in ((accn, num_hbm), (accd, den_hbm)):
        for kk in range(5):
            r0 = row0 + kk * 125
            pltpu.sync_copy(acc.at[pl.ds(r0, 125), :],
                            out.at[c, pl.ds(r0, 125), :])


def _make_sc_edge(emit_enew):
    outs = [jax.ShapeDtypeStruct((NC, N, H), F32),
            jax.ShapeDtypeStruct((NC, N, H), F32)]
    if emit_enew:
        outs.append(jax.ShapeDtypeStruct((NC, E, H), F32))
    mesh = plsc.VectorSubcoreMesh(core_axis_name="c", subcore_axis_name="s",
                                  num_cores=NC, num_subcores=NS)
    return pl.kernel(
        functools.partial(_sc_edge_body, emit_enew),
        out_type=tuple(outs),
        mesh=mesh,
        compiler_params=pltpu.CompilerParams(use_tc_tiling_on_sc=False),
        scratch_types=[
            pltpu.VMEM_SHARED((N, H), F32),
            pltpu.VMEM_SHARED((N, H), F32),
            pltpu.VMEM((CH,), jnp.int32),
            pltpu.VMEM((CH,), jnp.int32),
            pltpu.VMEM((CH,), jnp.int32),
            pltpu.VMEM((CH,), jnp.int32),
            pltpu.VMEM((CH,), jnp.int32),
            pltpu.VMEM((CH,), jnp.int32),
            pltpu.VMEM((CH, D), F32),
            pltpu.VMEM((CH, D), F32),
            pltpu.VMEM((CH, H), F32),
            pltpu.VMEM((CH, H), F32),
            pltpu.VMEM((CH, H), F32),
            pltpu.VMEM((CH, H), F32),
            pltpu.VMEM((CH, H), F32),
            pltpu.SemaphoreType.DMA,
            pltpu.SemaphoreType.DMA,
            pltpu.SemaphoreType.DMA,
            pltpu.SemaphoreType.DMA,
            pltpu.SemaphoreType.DMA,
            pltpu.SemaphoreType.DMA,
        ],
        name="sc_edge_pass" + ("_l1" if emit_enew else "_l2"),
    )


_sc_edge_l1 = _make_sc_edge(True)
_sc_edge_l2 = _make_sc_edge(False)


def _mm_body(x_ref, w_ref, b_ref, o_ref):
    o_ref[...] = jnp.dot(x_ref[...], w_ref[...],
                         preferred_element_type=F32) + b_ref[...]


def _mm(x, w, b):
    n, din = x.shape
    dout = w.shape[1]
    return pl.pallas_call(
        _mm_body,
        grid=(n // TB_N,),
        in_specs=[pl.BlockSpec((TB_N, din), lambda i: (i, 0)),
                  pl.BlockSpec((din, dout), lambda i: (0, 0)),
                  pl.BlockSpec((1, dout), lambda i: (0, 0))],
        out_specs=pl.BlockSpec((TB_N, dout), lambda i: (i, 0)),
        out_shape=jax.ShapeDtypeStruct((n, dout), F32),
    )(x, w, b.reshape(1, dout))


def _outer_body(ef_ref, uv_ref, o_ref):
    t = ef_ref[...] * uv_ref[0:1, :] + uv_ref[1:2, :]
    o_ref[0] = t[:, :H]
    o_ref[1] = t[:, H:]


def _outer(ef, u, v):
    uv = jnp.stack([u, v])
    return pl.pallas_call(
        _outer_body,
        grid=(E // TB_E,),
        in_specs=[pl.BlockSpec((TB_E, 1), lambda i: (i, 0)),
                  pl.BlockSpec((2, D), lambda i: (0, 0))],
        out_specs=pl.BlockSpec((NC, TB_E, H), lambda i: (0, i, 0)),
        out_shape=jax.ShapeDtypeStruct((NC, E, H), F32),
    )(ef, uv)


def _estats_body(en_ref, sn_ref, o_ref):
    @pl.when(pl.program_id(0) == 0)
    def _init():
        o_ref[...] = jnp.zeros_like(o_ref)
    eb = jnp.concatenate([en_ref[0], en_ref[1]], axis=1) * sn_ref[...]
    o_ref[...] += jnp.stack([jnp.sum(eb, axis=0), jnp.sum(eb * eb, axis=0)])


def _estats(enew, sne):
    return pl.pallas_call(
        _estats_body,
        grid=(E // TB_E,),
        in_specs=[pl.BlockSpec((NC, TB_E, H), lambda i: (0, i, 0)),
                  pl.BlockSpec((TB_E, 1), lambda i: (i, 0))],
        out_specs=pl.BlockSpec((2, D), lambda i: (0, 0)),
        out_shape=jax.ShapeDtypeStruct((2, D), F32),
    )(enew, sne)


def _ztrans_body(en_ref, sn_ref, ef_ref, c2_ref, aux_ref, o_ref):
    eb = jnp.concatenate([en_ref[0], en_ref[1]], axis=1) * sn_ref[...]
    r = jnp.maximum(eb * aux_ref[0:1, :] + aux_ref[1:2, :], 0.0)
    zc = (jnp.dot(r, c2_ref[...], preferred_element_type=F32)
          + ef_ref[...] * aux_ref[2:3, :] + aux_ref[3:4, :])
    o_ref[0] = zc[:, :H]
    o_ref[1] = zc[:, H:]


def _ztransform(enew, sne, ef, c2w, ge, bbe, u2, v2c):
    aux = jnp.stack([ge, bbe, u2, v2c])
    return pl.pallas_call(
        _ztrans_body,
        grid=(E // TB_E,),
        in_specs=[pl.BlockSpec((NC, TB_E, H), lambda i: (0, i, 0)),
                  pl.BlockSpec((TB_E, 1), lambda i: (i, 0)),
                  pl.BlockSpec((TB_E, 1), lambda i: (i, 0)),
                  pl.BlockSpec((D, D), lambda i: (0, 0)),
                  pl.BlockSpec((4, D), lambda i: (0, 0))],
        out_specs=pl.BlockSpec((NC, TB_E, H), lambda i: (0, i, 0)),
        out_shape=jax.ShapeDtypeStruct((NC, E, H), F32),
    )(enew, sne, ef, c2w, aux)


def _hstats_body(num_ref, den_ref, ah_ref, snn_ref, hb_ref, st_ref):
    @pl.when(pl.program_id(0) == 0)
    def _init():
        st_ref[...] = jnp.zeros_like(st_ref)
    num = jnp.concatenate([num_ref[0], num_ref[1]], axis=1)
    den = jnp.concatenate([den_ref[0], den_ref[1]], axis=1)
    m = (ah_ref[...] + num / (den + 1e-6)) * snn_ref[...]
    hb_ref[...] = m
    st_ref[...] += jnp.stack([jnp.sum(m, axis=0), jnp.sum(m * m, axis=0)])


def _hstats(num, den, ah, snn):
    return pl.pallas_call(
        _hstats_body,
        grid=(N // TB_N,),
        in_specs=[pl.BlockSpec((NC, TB_N, H), lambda i: (0, i, 0)),
                  pl.BlockSpec((NC, TB_N, H), lambda i: (0, i, 0)),
                  pl.BlockSpec((TB_N, D), lambda i: (i, 0)),
                  pl.BlockSpec((TB_N, 1), lambda i: (i, 0))],
        out_specs=[pl.BlockSpec((TB_N, D), lambda i: (i, 0)),
                   pl.BlockSpec((2, D), lambda i: (0, 0))],
        out_shape=[jax.ShapeDtypeStruct((N, D), F32),
                   jax.ShapeDtypeStruct((2, D), F32)],
    )(num, den, ah, snn)


def _happly_body(hp_ref, hb_ref, gb_ref, o_ref, cs_ref):
    @pl.when(pl.program_id(0) == 0)
    def _init():
        cs_ref[...] = jnp.zeros_like(cs_ref)
    hn = hp_ref[...] + jnp.maximum(
        hb_ref[...] * gb_ref[0:1, :] + gb_ref[1:2, :], 0.0)
    o_ref[...] = hn
    cs_ref[...] += jnp.sum(hn, axis=0, keepdims=True)


def _happly(hprev, hb, g, b):
    gb = jnp.stack([g, b])
    return pl.pallas_call(
        _happly_body,
        grid=(N // TB_N,),
        in_specs=[pl.BlockSpec((TB_N, D), lambda i: (i, 0)),
                  pl.BlockSpec((TB_N, D), lambda i: (i, 0)),
                  pl.BlockSpec((2, D), lambda i: (0, 0))],
        out_specs=[pl.BlockSpec((TB_N, D), lambda i: (i, 0)),
                   pl.BlockSpec((1, D), lambda i: (0, 0))],
        out_shape=[jax.ShapeDtypeStruct((N, D), F32),
                   jax.ShapeDtypeStruct((1, D), F32)],
    )(hprev, hb, gb)


def _bn_consts(stats, count, gamma, beta, eps=1e-5):
    mu = stats[0] / count
    var = stats[1] / count - mu * mu
    g = gamma / jnp.sqrt(var + eps)
    return g, beta - mu * g


def _split_tables(t4):
    ah = t4[:, :D]
    bh = t4[:, D:2 * D]
    dh = t4[:, 2 * D:3 * D]
    eh = t4[:, 3 * D:]
    db = jnp.concatenate([
        jnp.concatenate([dh[:, :H], bh[:, :H]], axis=1),
        jnp.concatenate([dh[:, H:], bh[:, H:]], axis=1)], axis=0)
    ehh = jnp.concatenate([eh[:, :H], eh[:, H:]], axis=0)
    return ah, db, ehh


def kernel(nodes_feat, edges_feat, nodes_num_norm_sqrt, edges_num_norm_sqrt,
           edge_index, params):
    ef = edges_feat
    sne = edges_num_norm_sqrt
    snn = nodes_num_norm_sqrt
    src = edge_index[0]
    dst = edge_index[1]
    sd = src + dst * 65536

    p1, p2 = params['layers']
    w_e = params['emb_e'][0][0]
    b_e = params['emb_e'][1]

    h0 = _mm(nodes_feat, params['emb_h'][0], params['emb_h'][1])

    def layer_tables(h, p):
        wcat = jnp.concatenate([p['A'][0], p['B'][0], p['D'][0], p['E'][0]],
                               axis=1)
        bcat = jnp.concatenate([p['A'][1], p['B'][1], p['D'][1], p['E'][1]])
        return _split_tables(_mm(h, wcat, bcat))

    ah1, db1, ehh1 = layer_tables(h0, p1)
    u1 = w_e @ p1['C'][0]
    v1 = b_e @ p1['C'][0] + p1['C'][1]
    base1 = _outer(ef, u1, v1)
    num1, den1, enew1 = _sc_edge_l1(db1, ehh1, base1, sd)
    hb1, hst1 = _hstats(num1, den1, ah1, snn)
    g1, b1 = _bn_consts(hst1, N, p1['bn_h'][0], p1['bn_h'][1])
    h1, _ = _happly(h0, hb1, g1, b1)

    est = _estats(enew1, sne)
    ge, bbe = _bn_consts(est, E, p1['bn_e'][0], p1['bn_e'][1])

    ah2, db2, ehh2 = layer_tables(h1, p2)
    u2 = w_e @ p2['C'][0]
    v2c = b_e @ p2['C'][0] + p2['C'][1]
    zc = _ztransform(enew1, sne, ef, p2['C'][0], ge, bbe, u2, v2c)
    num2, den2 = _sc_edge_l2(db2, ehh2, zc, sd)
    hb2, hst2 = _hstats(num2, den2, ah2, snn)
    g2, b2 = _bn_consts(hst2, N, p2['bn_h'][0], p2['bn_h'][1])
    _, colsum = _happly(h1, hb2, g2, b2)

    return colsum * (1.0 / N)


# speedup vs baseline: 2.2553x; 1.8001x over previous
import functools

import jax
import jax.numpy as jnp
from jax import lax
from jax.experimental import pallas as pl
from jax.experimental.pallas import tpu as pltpu
from jax.experimental.pallas import tpu_sc as plsc

N = 10000
E = 320000
D = 128
H = 64
NC = 2
NS = 16
TILE_E = E // NS
CH = 80
NCH = TILE_E // CH
NPAIR = NCH // 2
NROW = N // NS
TB_N = 400
TB_E = 512
F32 = jnp.float32


def _sc_edge_body(emit_enew, *refs):
    if emit_enew:
        (db_hbm, eh_hbm, base_hbm, sd_hbm,
         num_hbm, den_hbm, enew_hbm,
         accn, accd,
         sdb0, sdb1, idb0, idb1, ieb0, ieb1,
         dbb0, dbb1, ehb0, ehb1, bab0, bab1, enb,
         si0, si1, sgd0, sgd1, sge0, sge1) = refs
    else:
        (db_hbm, eh_hbm, base_hbm, sd_hbm,
         num_hbm, den_hbm,
         accn, accd,
         sdb0, sdb1, idb0, idb1, ieb0, ieb1,
         dbb0, dbb1, ehb0, ehb1, bab0, bab1, enb,
         si0, si1, sgd0, sgd1, sge0, sge1) = refs
        enew_hbm = None

    c = lax.axis_index("c")
    s = lax.axis_index("s")
    row0 = s * NROW
    ebase = s * TILE_E
    cN = c * N

    sdb = (sdb0, sdb1)
    idb = (idb0, idb1)
    ieb = (ieb0, ieb1)
    dbb = (dbb0, dbb1)
    ehb = (ehb0, ehb1)
    bab = (bab0, bab1)
    si = (si0, si1)
    sgd = (sgd0, sgd1)
    sge = (sge0, sge1)

    @plsc.parallel_loop(0, CH, step=1, unroll=4)
    def zfill(t):
        for j in range(4):
            ehb0[t, pl.ds(j * 16, 16)] = jnp.zeros((16,), F32)
    for acc in (accn, accd):
        for kk in range(7):
            pltpu.sync_copy(ehb0, acc.at[pl.ds(row0 + kk * CH, CH), :])
        pltpu.sync_copy(ehb0.at[pl.ds(0, NROW - 7 * CH)],
                        acc.at[pl.ds(row0 + 7 * CH, NROW - 7 * CH), :])
    plsc.subcore_barrier()

    def issue_in(k, b):
        e0 = ebase + k * CH
        pltpu.async_copy(sd_hbm.at[pl.ds(e0, CH)], sdb[b], si[b])
        pltpu.async_copy(base_hbm.at[c, pl.ds(e0, CH), :], bab[b], si[b])

    def wait_in(k, b):
        e0 = ebase + k * CH
        pltpu.make_async_copy(sd_hbm.at[pl.ds(e0, CH)], sdb[b], si[b]).wait()
        pltpu.make_async_copy(base_hbm.at[c, pl.ds(e0, CH), :], bab[b],
                              si[b]).wait()

    def mkidx(b):
        for j in range(CH // 16):
            sl = pl.ds(j * 16, 16)
            sd = sdb[b][sl]
            idb[b][sl] = (sd & 0xFFFF) + cN
            ieb[b][sl] = lax.shift_right_logical(sd, 16) + cN

    def issue_g(b):
        pltpu.async_copy(db_hbm.at[idb[b]], dbb[b], sgd[b])
        pltpu.async_copy(eh_hbm.at[ieb[b]], ehb[b], sge[b])

    def wait_g(b):
        pltpu.make_async_copy(db_hbm.at[idb[b]], dbb[b], sgd[b]).wait()
        pltpu.make_async_copy(eh_hbm.at[ieb[b]], ehb[b], sge[b]).wait()

    def compute(b):
        dbbb, ehbb, babb = dbb[b], ehb[b], bab[b]

        @plsc.parallel_loop(0, CH, step=1, unroll=4)
        def edge(t):
            for j in range(4):
                fl = pl.ds(j * 16, 16)
                z = babb[t, fl] + dbbb[t, fl] + ehbb[t, fl]
                if emit_enew:
                    enb[t, fl] = z
                sg = 1.0 / (1.0 + jnp.exp(-z))
                ehbb[t, fl] = sg * dbbb[t, pl.ds(64 + j * 16, 16)]
                babb[t, fl] = sg

    def finish(k, b):
        for j in range(CH // 16):
            sl = pl.ds(j * 16, 16)
            ieb[b][sl] = ieb[b][sl] - cN
        if emit_enew:
            e0 = ebase + k * CH
            pltpu.sync_copy(enb, enew_hbm.at[c, pl.ds(e0, CH), :])
        pltpu.sync_copy(ehb[b], accn.at[ieb[b]], add=True)
        pltpu.sync_copy(bab[b], accd.at[ieb[b]], add=True)

    issue_in(0, 0)
    issue_in(1, 1)
    wait_in(0, 0)
    mkidx(0)
    issue_g(0)

    def pair(p, _):
        k0 = 2 * p
        k1 = k0 + 1
        wait_g(0)
        wait_in(k1, 1)
        mkidx(1)
        issue_g(1)
        compute(0)
        finish(k0, 0)

        @pl.when(p < NPAIR - 1)
        def _in0():
            issue_in(k0 + 2, 0)

        wait_g(1)

        @pl.when(p < NPAIR - 1)
        def _g0():
            wait_in(k0 + 2, 0)
            mkidx(0)
            issue_g(0)
        compute(1)
        finish(k1, 1)

        @pl.when(p < NPAIR - 1)
        def _in1():
            issue_in(k1 + 2, 1)
        return 0

    lax.fori_loop(0, NPAIR, pair, 0)

    plsc.subcore_barrier()
    for acc, out in ((accn, num_hbm), (accd, den_hbm)):
        for kk in range(5):
            r0 = row0 + kk * 125
            pltpu.sync_copy(acc.at[pl.ds(r0, 125), :],
                            out.at[c, pl.ds(r0, 125), :])


def _make_sc_edge(emit_enew):
    outs = [jax.ShapeDtypeStruct((NC, N, H), F32),
            jax.ShapeDtypeStruct((NC, N, H), F32)]
    if emit_enew:
        outs.append(jax.ShapeDtypeStruct((NC, E, H), F32))
    mesh = plsc.VectorSubcoreMesh(core_axis_name="c", subcore_axis_name="s",
                                  num_cores=NC, num_subcores=NS)
    return pl.kernel(
        functools.partial(_sc_edge_body, emit_enew),
        out_type=tuple(outs),
        mesh=mesh,
        compiler_params=pltpu.CompilerParams(use_tc_tiling_on_sc=False),
        scratch_types=[
            pltpu.VMEM_SHARED((N, H), F32),
            pltpu.VMEM_SHARED((N, H), F32),
            pltpu.VMEM((CH,), jnp.int32),
            pltpu.VMEM((CH,), jnp.int32),
            pltpu.VMEM((CH,), jnp.int32),
            pltpu.VMEM((CH,), jnp.int32),
            pltpu.VMEM((CH,), jnp.int32),
            pltpu.VMEM((CH,), jnp.int32),
            pltpu.VMEM((CH, D), F32),
            pltpu.VMEM((CH, D), F32),
            pltpu.VMEM((CH, H), F32),
            pltpu.VMEM((CH, H), F32),
            pltpu.VMEM((CH, H), F32),
            pltpu.VMEM((CH, H), F32),
            pltpu.VMEM((CH, H), F32),
            pltpu.SemaphoreType.DMA,
            pltpu.SemaphoreType.DMA,
            pltpu.SemaphoreType.DMA,
            pltpu.SemaphoreType.DMA,
            pltpu.SemaphoreType.DMA,
            pltpu.SemaphoreType.DMA,
        ],
        name="sc_edge_pass" + ("_l1" if emit_enew else "_l2"),
    )


_sc_edge_l1 = _make_sc_edge(True)
_sc_edge_l2 = _make_sc_edge(False)


def _mm_body(x_ref, w_ref, b_ref, o_ref):
    o_ref[...] = jnp.dot(x_ref[...], w_ref[...],
                         preferred_element_type=F32) + b_ref[...]


def _mm(x, w, b):
    n, din = x.shape
    dout = w.shape[1]
    return pl.pallas_call(
        _mm_body,
        grid=(n // TB_N,),
        in_specs=[pl.BlockSpec((TB_N, din), lambda i: (i, 0)),
                  pl.BlockSpec((din, dout), lambda i: (0, 0)),
                  pl.BlockSpec((1, dout), lambda i: (0, 0))],
        out_specs=pl.BlockSpec((TB_N, dout), lambda i: (i, 0)),
        out_shape=jax.ShapeDtypeStruct((n, dout), F32),
    )(x, w, b.reshape(1, dout))


def _outer_body(ef_ref, uv_ref, o_ref):
    t = ef_ref[...] * uv_ref[0:1, :] + uv_ref[1:2, :]
    o_ref[0] = t[:, :H]
    o_ref[1] = t[:, H:]


def _outer(ef, u, v):
    uv = jnp.stack([u, v])
    return pl.pallas_call(
        _outer_body,
        grid=(E // TB_E,),
        in_specs=[pl.BlockSpec((TB_E, 1), lambda i: (i, 0)),
                  pl.BlockSpec((2, D), lambda i: (0, 0))],
        out_specs=pl.BlockSpec((NC, TB_E, H), lambda i: (0, i, 0)),
        out_shape=jax.ShapeDtypeStruct((NC, E, H), F32),
    )(ef, uv)


def _estats_body(en_ref, sn_ref, o_ref):
    @pl.when(pl.program_id(0) == 0)
    def _init():
        o_ref[...] = jnp.zeros_like(o_ref)
    eb = jnp.concatenate([en_ref[0], en_ref[1]], axis=1) * sn_ref[...]
    o_ref[...] += jnp.stack([jnp.sum(eb, axis=0), jnp.sum(eb * eb, axis=0)])


def _estats(enew, sne):
    return pl.pallas_call(
        _estats_body,
        grid=(E // TB_E,),
        in_specs=[pl.BlockSpec((NC, TB_E, H), lambda i: (0, i, 0)),
                  pl.BlockSpec((TB_E, 1), lambda i: (i, 0))],
        out_specs=pl.BlockSpec((2, D), lambda i: (0, 0)),
        out_shape=jax.ShapeDtypeStruct((2, D), F32),
    )(enew, sne)


def _ztrans_body(en_ref, sn_ref, ef_ref, c2_ref, aux_ref, o_ref):
    eb = jnp.concatenate([en_ref[0], en_ref[1]], axis=1) * sn_ref[...]
    r = jnp.maximum(eb * aux_ref[0:1, :] + aux_ref[1:2, :], 0.0)
    zc = (jnp.dot(r, c2_ref[...], preferred_element_type=F32)
          + ef_ref[...] * aux_ref[2:3, :] + aux_ref[3:4, :])
    o_ref[0] = zc[:, :H]
    o_ref[1] = zc[:, H:]


def _ztransform(enew, sne, ef, c2w, ge, bbe, u2, v2c):
    aux = jnp.stack([ge, bbe, u2, v2c])
    return pl.pallas_call(
        _ztrans_body,
        grid=(E // TB_E,),
        in_specs=[pl.BlockSpec((NC, TB_E, H), lambda i: (0, i, 0)),
                  pl.BlockSpec((TB_E, 1), lambda i: (i, 0)),
                  pl.BlockSpec((TB_E, 1), lambda i: (i, 0)),
                  pl.BlockSpec((D, D), lambda i: (0, 0)),
                  pl.BlockSpec((4, D), lambda i: (0, 0))],
        out_specs=pl.BlockSpec((NC, TB_E, H), lambda i: (0, i, 0)),
        out_shape=jax.ShapeDtypeStruct((NC, E, H), F32),
    )(enew, sne, ef, c2w, aux)


def _hstats_body(num_ref, den_ref, ah_ref, snn_ref, hb_ref, st_ref):
    @pl.when(pl.program_id(0) == 0)
    def _init():
        st_ref[...] = jnp.zeros_like(st_ref)
    num = jnp.concatenate([num_ref[0], num_ref[1]], axis=1)
    den = jnp.concatenate([den_ref[0], den_ref[1]], axis=1)
    m = (ah_ref[...] + num / (den + 1e-6)) * snn_ref[...]
    hb_ref[...] = m
    st_ref[...] += jnp.stack([jnp.sum(m, axis=0), jnp.sum(m * m, axis=0)])


def _hstats(num, den, ah, snn):
    return pl.pallas_call(
        _hstats_body,
        grid=(N // TB_N,),
        in_specs=[pl.BlockSpec((NC, TB_N, H), lambda i: (0, i, 0)),
                  pl.BlockSpec((NC, TB_N, H), lambda i: (0, i, 0)),
                  pl.BlockSpec((TB_N, D), lambda i: (i, 0)),
                  pl.BlockSpec((TB_N, 1), lambda i: (i, 0))],
        out_specs=[pl.BlockSpec((TB_N, D), lambda i: (i, 0)),
                   pl.BlockSpec((2, D), lambda i: (0, 0))],
        out_shape=[jax.ShapeDtypeStruct((N, D), F32),
                   jax.ShapeDtypeStruct((2, D), F32)],
    )(num, den, ah, snn)


def _happly_body(hp_ref, hb_ref, gb_ref, o_ref, cs_ref):
    @pl.when(pl.program_id(0) == 0)
    def _init():
        cs_ref[...] = jnp.zeros_like(cs_ref)
    hn = hp_ref[...] + jnp.maximum(
        hb_ref[...] * gb_ref[0:1, :] + gb_ref[1:2, :], 0.0)
    o_ref[...] = hn
    cs_ref[...] += jnp.sum(hn, axis=0, keepdims=True)


def _happly(hprev, hb, g, b):
    gb = jnp.stack([g, b])
    return pl.pallas_call(
        _happly_body,
        grid=(N // TB_N,),
        in_specs=[pl.BlockSpec((TB_N, D), lambda i: (i, 0)),
                  pl.BlockSpec((TB_N, D), lambda i: (i, 0)),
                  pl.BlockSpec((2, D), lambda i: (0, 0))],
        out_specs=[pl.BlockSpec((TB_N, D), lambda i: (i, 0)),
                   pl.BlockSpec((1, D), lambda i: (0, 0))],
        out_shape=[jax.ShapeDtypeStruct((N, D), F32),
                   jax.ShapeDtypeStruct((1, D), F32)],
    )(hprev, hb, gb)


def _bn_consts(stats, count, gamma, beta, eps=1e-5):
    mu = stats[0] / count
    var = stats[1] / count - mu * mu
    g = gamma / jnp.sqrt(var + eps)
    return g, beta - mu * g


def _split_tables(t4):
    ah = t4[:, :D]
    bh = t4[:, D:2 * D]
    dh = t4[:, 2 * D:3 * D]
    eh = t4[:, 3 * D:]
    db = jnp.concatenate([
        jnp.concatenate([dh[:, :H], bh[:, :H]], axis=1),
        jnp.concatenate([dh[:, H:], bh[:, H:]], axis=1)], axis=0)
    ehh = jnp.concatenate([eh[:, :H], eh[:, H:]], axis=0)
    return ah, db, ehh


def kernel(nodes_feat, edges_feat, nodes_num_norm_sqrt, edges_num_norm_sqrt,
           edge_index, params):
    ef = edges_feat
    sne = edges_num_norm_sqrt
    snn = nodes_num_norm_sqrt
    src = edge_index[0]
    dst = edge_index[1]
    sd = src + dst * 65536

    p1, p2 = params['layers']
    w_e = params['emb_e'][0][0]
    b_e = params['emb_e'][1]

    h0 = _mm(nodes_feat, params['emb_h'][0], params['emb_h'][1])

    def layer_tables(h, p):
        wcat = jnp.concatenate([p['A'][0], p['B'][0], p['D'][0], p['E'][0]],
                               axis=1)
        bcat = jnp.concatenate([p['A'][1], p['B'][1], p['D'][1], p['E'][1]])
        return _split_tables(_mm(h, wcat, bcat))

    ah1, db1, ehh1 = layer_tables(h0, p1)
    u1 = w_e @ p1['C'][0]
    v1 = b_e @ p1['C'][0] + p1['C'][1]
    base1 = _outer(ef, u1, v1)
    num1, den1, enew1 = _sc_edge_l1(db1, ehh1, base1, sd)
    hb1, hst1 = _hstats(num1, den1, ah1, snn)
    g1, b1 = _bn_consts(hst1, N, p1['bn_h'][0], p1['bn_h'][1])
    h1, _ = _happly(h0, hb1, g1, b1)

    est = _estats(enew1, sne)
    ge, bbe = _bn_consts(est, E, p1['bn_e'][0], p1['bn_e'][1])

    ah2, db2, ehh2 = layer_tables(h1, p2)
    u2 = w_e @ p2['C'][0]
    v2c = b_e @ p2['C'][0] + p2['C'][1]
    zc = _ztransform(enew1, sne, ef, p2['C'][0], ge, bbe, u2, v2c)
    num2, den2 = _sc_edge_l2(db2, ehh2, zc, sd)
    hb2, hst2 = _hstats(num2, den2, ah2, snn)
    g2, b2 = _bn_consts(hst2, N, p2['bn_h'][0], p2['bn_h'][1])
    _, colsum = _happly(h1, hb2, g2, b2)

    return colsum * (1.0 / N)
